# baseline (speedup 1.0000x reference)
import functools

import jax
import jax.numpy as jnp
from jax import lax
from jax.experimental import pallas as pl
from jax.experimental.pallas import tpu as pltpu
from jax.experimental.pallas import tpu_sc as plsc

N = 10000
E = 320000
D = 128

NC = 2
NS = 16
NW = NC * NS
EPW = E // NW
CH = 128
NCHUNK = 80
EPWP = NCHUNK * CH
RING = 2
SLAB = 8
NSLAB = NCHUNK // SLAB
EP = EPWP * NW
NPAD = 10240
ROWS_PT = NPAD // NS

def _seg_body(with_counts, table, edges, *rest):
    _Z16 = jnp.zeros((16,), jnp.float32)
    _O16 = jnp.ones((16,), jnp.float32)
    rest = list(rest)
    out = rest.pop(0)
    cnt = rest.pop(0) if with_counts else None
    acc = rest.pop(0)
    cntacc = rest.pop(0) if with_counts else None
    ssb = [rest.pop(0), rest.pop(0)]
    dsb = [rest.pop(0), rest.pop(0)]
    rows = [rest.pop(0) for _ in range(RING)]
    onesv = rest.pop(0) if with_counts else None
    zc = rest.pop(0) if with_counts else None
    gsem = [rest.pop(0) for _ in range(RING)]
    ssem = [rest.pop(0) for _ in range(RING)]
    slsem = [rest.pop(0), rest.pop(0)]
    r0 = rows[0]
    c = lax.axis_index("c")
    s = lax.axis_index("s")
    w = s * NC + c

    @pl.loop(0, CH)
    def _zrow(i):
        for j in range(D // 16):
            r0[i, pl.ds(j * 16, 16)] = _Z16

    for j in range(ROWS_PT // CH):
        pltpu.sync_copy(r0, acc.at[pl.ds(s * ROWS_PT + j * CH, CH)])

    if with_counts:
        @pl.loop(0, ROWS_PT // 16)
        def _zcrow(i):
            zc[pl.ds(i * 16, 16)] = _Z16

        pltpu.sync_copy(zc, cntacc.at[pl.ds(s * ROWS_PT, ROWS_PT)])
        for j in range(CH // 16):
            onesv[pl.ds(j * 16, 16)] = _O16

    plsc.subcore_barrier()

    def load_slab(sl, ss):
        pltpu.async_copy(edges.at[0, w, sl], ssb[ss], slsem[ss])
        pltpu.async_copy(edges.at[1, w, sl], dsb[ss], slsem[ss])

    def wait_slab(ss):
        pltpu.make_async_copy(edges.at[0, w, 0], ssb[ss], slsem[ss]).wait()
        pltpu.make_async_copy(edges.at[1, w, 0], dsb[ss], slsem[ss]).wait()

    def start_gather(ss, j, b):
        pltpu.async_copy(table.at[ssb[ss].at[j]], rows[b], gsem[b])

    def wait_gather(ss, j, b):
        pltpu.make_async_copy(table.at[ssb[ss].at[j]], rows[b], gsem[b]).wait()

    def drain_scatter(ss, j, b):
        pltpu.make_async_copy(rows[b], acc.at[dsb[ss].at[j]], ssem[b]).wait()
        if with_counts:
            pltpu.make_async_copy(onesv, cntacc.at[dsb[ss].at[j]],
                                  ssem[b]).wait()

    def chunk(k, ss, j):
        b = j % RING
        nb = (b + 1) % RING
        ss1, j1 = (ss, j + 1) if j + 1 < SLAB else (1 - ss, 0)
        ssp, jp = (ss, j - 1) if j > 0 else (1 - ss, SLAB - 1)
        wait_gather(ss, j, b)

        @pl.when(k >= 1)
        def _():
            drain_scatter(ssp, jp, nb)

        @pl.when(k + 1 < NCHUNK)
        def _():
            if j1 == 0:
                wait_slab(ss1)
            start_gather(ss1, j1, nb)

        pltpu.async_copy(rows[b], acc.at[dsb[ss].at[j]], ssem[b], add=True)
        if with_counts:
            pltpu.async_copy(onesv, cntacc.at[dsb[ss].at[j]], ssem[b],
                             add=True)

        if j == 1:
            @pl.when((k >= SLAB) & (k < (NSLAB - 1) * SLAB))
            def _():
                load_slab(k // SLAB + 1, 1 - ss)

    load_slab(0, 0)
    load_slab(1, 1)
    wait_slab(0)
    start_gather(0, 0, 0)

    @pl.loop(0, NSLAB // 2)
    def _step(i):
        for ss in range(2):
            for j in range(SLAB):
                chunk((2 * i + ss) * SLAB + j, ss, j)

    drain_scatter(1, SLAB - 1, (SLAB - 1) % RING)

    plsc.subcore_barrier()

    pltpu.sync_copy(acc.at[pl.ds(s * ROWS_PT, ROWS_PT)],
                    out.at[c, pl.ds(s * ROWS_PT, ROWS_PT)])
    if with_counts:
        pltpu.sync_copy(cntacc.at[pl.ds(s * ROWS_PT, ROWS_PT)],
                        cnt.at[pl.ds(c * NPAD + s * ROWS_PT, ROWS_PT)])


def _make_seg(with_counts):
    out_type = [jax.ShapeDtypeStruct((NC, NPAD, D), jnp.float32)]
    scratch = [
        pltpu.VMEM_SHARED((NPAD, D), jnp.float32),
    ]
    if with_counts:
        out_type.append(jax.ShapeDtypeStruct((NC * NPAD,), jnp.float32))
        scratch.append(pltpu.VMEM_SHARED((NPAD,), jnp.float32))
    scratch += [pltpu.VMEM((SLAB, CH), jnp.int32) for _ in range(4)]
    scratch += [pltpu.VMEM((CH, D), jnp.float32) for _ in range(RING)]
    if with_counts:
        scratch += [pltpu.VMEM((CH,), jnp.float32),
                    pltpu.VMEM((ROWS_PT,), jnp.float32)]
    scratch += [pltpu.SemaphoreType.DMA for _ in range(2 * RING + 2)]
    return pl.kernel(
        functools.partial(_seg_body, with_counts),
        out_type=out_type,
        mesh=plsc.VectorSubcoreMesh(core_axis_name="c", subcore_axis_name="s",
                                    num_cores=NC, num_subcores=NS),
        scratch_types=scratch,
    )


_make_seg = functools.cache(_make_seg)

BR = 1000
_HI = jax.lax.Precision.DEFAULT


def _dot_t(x, w):
    return jax.lax.dot_general(x, w, (((1,), (1,)), ((), ())), precision=_HI)


def _mm_body(x_ref, w_ref, b_ref, o_ref):
    o_ref[...] = _dot_t(x_ref[...], w_ref[...]) + b_ref[...]


def _mid_body(p_ref, cnt_ref, xr_ref, w_ref, b_ref, o_ref):
    sblk = p_ref[0] + p_ref[1]
    c2 = cnt_ref[...]
    inv = 1.0 / jnp.maximum(c2[:, 0:1] + c2[:, 1:2], 1.0)
    h = jnp.maximum(sblk * inv + xr_ref[...], 0.0)
    o_ref[...] = _dot_t(h, w_ref[...]) + b_ref[...]


def _fin_body(q_ref, cnt_ref, hr_ref, o_ref):
    sblk = q_ref[0] + q_ref[1]
    c2 = cnt_ref[...]
    inv = 1.0 / jnp.maximum(c2[:, 0:1] + c2[:, 1:2], 1.0)
    o_ref[...] = sblk * inv + hr_ref[...]


_row_spec = pl.BlockSpec((BR, D), lambda i: (i, 0))
_p_spec = pl.BlockSpec((NC, BR, D), lambda i: (0, i, 0))
_cnt_spec = pl.BlockSpec((BR, 2), lambda i: (i, 0))
_w_spec = pl.BlockSpec((D, D), lambda i: (0, 0))
_b_spec = pl.BlockSpec((1, D), lambda i: (0, 0))

_mm = pl.pallas_call(
    _mm_body,
    grid=(N // BR,),
    in_specs=[_row_spec, _w_spec, _b_spec],
    out_specs=_row_spec,
    out_shape=jax.ShapeDtypeStruct((N, D), jnp.float32),
)

_mid = pl.pallas_call(
    _mid_body,
    grid=(N // BR,),
    in_specs=[_p_spec, _cnt_spec, _row_spec, _w_spec, _b_spec],
    out_specs=_row_spec,
    out_shape=jax.ShapeDtypeStruct((N, D), jnp.float32),
)

_fin = pl.pallas_call(
    _fin_body,
    grid=(N // BR,),
    in_specs=[_p_spec, _cnt_spec, _row_spec],
    out_specs=_row_spec,
    out_shape=jax.ShapeDtypeStruct((N, D), jnp.float32),
)


def kernel(x, edge_index, W1_l, b1, W1_r, W2_l, b2, W2_r):
    ei = edge_index.astype(jnp.int32).reshape(2, NW, EPW)
    npe = EPWP - EPW
    ar = jnp.arange(NW * npe, dtype=jnp.int32)
    pad = jnp.stack([(ar % N).reshape(NW, npe),
                     (N + ar % (NPAD - N)).reshape(NW, npe)])
    edges = jnp.concatenate([ei, pad], axis=2).reshape(2, NW, NSLAB, SLAB, CH)
    z = jnp.zeros((1, D), jnp.float32)
    xl1 = _mm(x, W1_l, z)
    p, cnt = _make_seg(True)(xl1, edges)
    xr1 = _mm(x, W1_r, b1.reshape(1, D))
    cntT = cnt.reshape(NC, NPAD)[:, :N].T
    hl2 = _mid(p, cntT, xr1, W2_l, z)
    q, = _make_seg(False)(hl2, edges)
    hr2 = _mid(p, cntT, xr1, W2_r, b2.reshape(1, D))
    return _fin(q, cntT, hr2)


# speedup vs baseline: 1.0174x; 1.0174x over previous
import functools

import jax
import jax.numpy as jnp
from jax import lax
from jax.experimental import pallas as pl
from jax.experimental.pallas import tpu as pltpu
from jax.experimental.pallas import tpu_sc as plsc

N = 10000
E = 320000
D = 128

NC = 2
NS = 16
NW = NC * NS
EPW = E // NW
CH = 128
NCHUNK = 80
EPWP = NCHUNK * CH
RING = 2
SLAB = 8
NSLAB = NCHUNK // SLAB
EP = EPWP * NW
NPAD = 10240
ROWS_PT = NPAD // NS

def _seg_body(with_counts, table, edges, *rest):
    _Z16 = jnp.zeros((16,), jnp.float32)
    _O16 = jnp.ones((16,), jnp.float32)
    rest = list(rest)
    out = rest.pop(0)
    cnt = rest.pop(0) if with_counts else None
    acc = rest.pop(0)
    cntacc = rest.pop(0) if with_counts else None
    ssb = [rest.pop(0), rest.pop(0)]
    dsb = [rest.pop(0), rest.pop(0)]
    rows = [rest.pop(0) for _ in range(RING)]
    onesv = rest.pop(0) if with_counts else None
    zc = rest.pop(0) if with_counts else None
    gsem = [rest.pop(0) for _ in range(RING)]
    ssem = [rest.pop(0) for _ in range(RING)]
    slsem = [rest.pop(0), rest.pop(0)]
    r0 = rows[0]
    c = lax.axis_index("c")
    s = lax.axis_index("s")
    w = s * NC + c

    @pl.loop(0, CH)
    def _zrow(i):
        for j in range(D // 16):
            r0[i, pl.ds(j * 16, 16)] = _Z16

    for j in range(ROWS_PT // CH):
        pltpu.sync_copy(r0, acc.at[pl.ds(s * ROWS_PT + j * CH, CH)])

    if with_counts:
        @pl.loop(0, ROWS_PT // 16)
        def _zcrow(i):
            zc[pl.ds(i * 16, 16)] = _Z16

        pltpu.sync_copy(zc, cntacc.at[pl.ds(s * ROWS_PT, ROWS_PT)])
        for j in range(CH // 16):
            onesv[pl.ds(j * 16, 16)] = _O16

    plsc.subcore_barrier()

    def load_slab(sl, ss):
        pltpu.async_copy(edges.at[0, w, sl], ssb[ss], slsem[ss])
        pltpu.async_copy(edges.at[1, w, sl], dsb[ss], slsem[ss])

    def wait_slab(ss):
        pltpu.make_async_copy(edges.at[0, w, 0], ssb[ss], slsem[ss]).wait()
        pltpu.make_async_copy(edges.at[1, w, 0], dsb[ss], slsem[ss]).wait()

    def start_gather(ss, j, b):
        pltpu.async_copy(table.at[ssb[ss].at[j]], rows[b], gsem[b])

    def wait_gather(ss, j, b):
        pltpu.make_async_copy(table.at[ssb[ss].at[j]], rows[b], gsem[b]).wait()

    def drain_scatter(ss, j, b):
        pltpu.make_async_copy(rows[b], acc.at[dsb[ss].at[j]], ssem[b]).wait()
        if with_counts:
            pltpu.make_async_copy(onesv, cntacc.at[dsb[ss].at[j]],
                                  ssem[b]).wait()

    def chunk(k, ss, j):
        b = j % RING
        nb = (b + 1) % RING
        ss1, j1 = (ss, j + 1) if j + 1 < SLAB else (1 - ss, 0)
        ssp, jp = (ss, j - 1) if j > 0 else (1 - ss, SLAB - 1)
        wait_gather(ss, j, b)

        @pl.when(k >= 1)
        def _():
            drain_scatter(ssp, jp, nb)

        @pl.when(k + 1 < NCHUNK)
        def _():
            if j1 == 0:
                wait_slab(ss1)
            start_gather(ss1, j1, nb)

        pltpu.async_copy(rows[b], acc.at[dsb[ss].at[j]], ssem[b], add=True)
        if with_counts:
            pltpu.async_copy(onesv, cntacc.at[dsb[ss].at[j]], ssem[b],
                             add=True)

        if j == 1:
            @pl.when((k >= SLAB) & (k < (NSLAB - 1) * SLAB))
            def _():
                load_slab(k // SLAB + 1, 1 - ss)

    load_slab(0, 0)
    load_slab(1, 1)
    wait_slab(0)
    start_gather(0, 0, 0)

    @pl.loop(0, NSLAB // 2)
    def _step(i):
        for ss in range(2):
            for j in range(SLAB):
                chunk((2 * i + ss) * SLAB + j, ss, j)

    drain_scatter(1, SLAB - 1, (SLAB - 1) % RING)

    plsc.subcore_barrier()

    pltpu.sync_copy(acc.at[pl.ds(s * ROWS_PT, ROWS_PT)],
                    out.at[c, pl.ds(s * ROWS_PT, ROWS_PT)])
    if with_counts:
        pltpu.sync_copy(cntacc.at[pl.ds(s * ROWS_PT, ROWS_PT)],
                        cnt.at[pl.ds(c * NPAD + s * ROWS_PT, ROWS_PT)])


def _make_seg(with_counts):
    out_type = [jax.ShapeDtypeStruct((NC, NPAD, D), jnp.float32)]
    scratch = [
        pltpu.VMEM_SHARED((NPAD, D), jnp.float32),
    ]
    if with_counts:
        out_type.append(jax.ShapeDtypeStruct((NC * NPAD,), jnp.float32))
        scratch.append(pltpu.VMEM_SHARED((NPAD,), jnp.float32))
    scratch += [pltpu.VMEM((SLAB, CH), jnp.int32) for _ in range(4)]
    scratch += [pltpu.VMEM((CH, D), jnp.float32) for _ in range(RING)]
    if with_counts:
        scratch += [pltpu.VMEM((CH,), jnp.float32),
                    pltpu.VMEM((ROWS_PT,), jnp.float32)]
    scratch += [pltpu.SemaphoreType.DMA for _ in range(2 * RING + 2)]
    return pl.kernel(
        functools.partial(_seg_body, with_counts),
        out_type=out_type,
        mesh=plsc.VectorSubcoreMesh(core_axis_name="c", subcore_axis_name="s",
                                    num_cores=NC, num_subcores=NS),
        scratch_types=scratch,
    )


_make_seg = functools.cache(_make_seg)

BR = 2000
_HI = jax.lax.Precision.DEFAULT


def _dot_t(x, w):
    return jax.lax.dot_general(x, w, (((1,), (1,)), ((), ())), precision=_HI)


def _mm_body(x_ref, w_ref, b_ref, o_ref):
    o_ref[...] = _dot_t(x_ref[...], w_ref[...]) + b_ref[...]


def _mid_body(p_ref, cnt_ref, xr_ref, w_ref, b_ref, o_ref):
    sblk = p_ref[0] + p_ref[1]
    c2 = cnt_ref[...]
    inv = 1.0 / jnp.maximum(c2[:, 0:1] + c2[:, 1:2], 1.0)
    h = jnp.maximum(sblk * inv + xr_ref[...], 0.0)
    o_ref[...] = _dot_t(h, w_ref[...]) + b_ref[...]


def _fin_body(q_ref, cnt_ref, hr_ref, o_ref):
    sblk = q_ref[0] + q_ref[1]
    c2 = cnt_ref[...]
    inv = 1.0 / jnp.maximum(c2[:, 0:1] + c2[:, 1:2], 1.0)
    o_ref[...] = sblk * inv + hr_ref[...]


_row_spec = pl.BlockSpec((BR, D), lambda i: (i, 0))
_p_spec = pl.BlockSpec((NC, BR, D), lambda i: (0, i, 0))
_cnt_spec = pl.BlockSpec((BR, 2), lambda i: (i, 0))
_w_spec = pl.BlockSpec((D, D), lambda i: (0, 0))
_b_spec = pl.BlockSpec((1, D), lambda i: (0, 0))

_mm = pl.pallas_call(
    _mm_body,
    grid=(N // BR,),
    in_specs=[_row_spec, _w_spec, _b_spec],
    out_specs=_row_spec,
    out_shape=jax.ShapeDtypeStruct((N, D), jnp.float32),
)

_mid = pl.pallas_call(
    _mid_body,
    grid=(N // BR,),
    in_specs=[_p_spec, _cnt_spec, _row_spec, _w_spec, _b_spec],
    out_specs=_row_spec,
    out_shape=jax.ShapeDtypeStruct((N, D), jnp.float32),
)

_fin = pl.pallas_call(
    _fin_body,
    grid=(N // BR,),
    in_specs=[_p_spec, _cnt_spec, _row_spec],
    out_specs=_row_spec,
    out_shape=jax.ShapeDtypeStruct((N, D), jnp.float32),
)


def kernel(x, edge_index, W1_l, b1, W1_r, W2_l, b2, W2_r):
    ei = edge_index.astype(jnp.int32).reshape(2, NW, EPW)
    npe = EPWP - EPW
    ar = jnp.arange(NW * npe, dtype=jnp.int32)
    pad = jnp.stack([(ar % N).reshape(NW, npe),
                     (N + ar % (NPAD - N)).reshape(NW, npe)])
    edges = jnp.concatenate([ei, pad], axis=2).reshape(2, NW, NSLAB, SLAB, CH)
    z = jnp.zeros((1, D), jnp.float32)
    xl1 = _mm(x, W1_l, z)
    p, cnt = _make_seg(True)(xl1, edges)
    xr1 = _mm(x, W1_r, b1.reshape(1, D))
    cntT = cnt.reshape(NC, NPAD)[:, :N].T
    hl2 = _mid(p, cntT, xr1, W2_l, z)
    q, = _make_seg(False)(hl2, edges)
    hr2 = _mid(p, cntT, xr1, W2_r, b2.reshape(1, D))
    return _fin(q, cntT, hr2)


# speedup vs baseline: 1.0175x; 1.0001x over previous
import functools

import jax
import jax.numpy as jnp
from jax import lax
from jax.experimental import pallas as pl
from jax.experimental.pallas import tpu as pltpu
from jax.experimental.pallas import tpu_sc as plsc

N = 10000
E = 320000
D = 128

NC = 2
NS = 16
NW = NC * NS
EPW = E // NW
CH = 128
NCHUNK = 80
EPWP = NCHUNK * CH
RING = 2
SLAB = 8
NSLAB = NCHUNK // SLAB
NPAD = 10240
ROWS_PT = NPAD // NS

def _seg_body(with_counts, table, edges, *rest):
    _Z16 = jnp.zeros((16,), jnp.float32)
    _O16 = jnp.ones((16,), jnp.float32)
    rest = list(rest)
    out = rest.pop(0)
    cnt = rest.pop(0) if with_counts else None
    acc = rest.pop(0)
    cntacc = rest.pop(0) if with_counts else None
    ssb = [rest.pop(0), rest.pop(0)]
    dsb = [rest.pop(0), rest.pop(0)]
    rows = [rest.pop(0) for _ in range(RING)]
    onesv = rest.pop(0) if with_counts else None
    zc = rest.pop(0) if with_counts else None
    gsem = [rest.pop(0) for _ in range(RING)]
    ssem = [rest.pop(0) for _ in range(RING)]
    slsem = [rest.pop(0), rest.pop(0)]
    r0 = rows[0]
    c = lax.axis_index("c")
    s = lax.axis_index("s")
    w = s * NC + c

    @pl.loop(0, CH)
    def _zrow(i):
        for j in range(D // 16):
            r0[i, pl.ds(j * 16, 16)] = _Z16

    for j in range(ROWS_PT // CH):
        pltpu.sync_copy(r0, acc.at[pl.ds(s * ROWS_PT + j * CH, CH)])

    if with_counts:
        @pl.loop(0, ROWS_PT // 16)
        def _zcrow(i):
            zc[pl.ds(i * 16, 16)] = _Z16

        pltpu.sync_copy(zc, cntacc.at[pl.ds(s * ROWS_PT, ROWS_PT)])
        for j in range(CH // 16):
            onesv[pl.ds(j * 16, 16)] = _O16

    plsc.subcore_barrier()

    def load_slab(sl, ss):
        pltpu.async_copy(edges.at[0, w, sl], ssb[ss], slsem[ss])
        pltpu.async_copy(edges.at[1, w, sl], dsb[ss], slsem[ss])

    def wait_slab(ss):
        pltpu.make_async_copy(edges.at[0, w, 0], ssb[ss], slsem[ss]).wait()
        pltpu.make_async_copy(edges.at[1, w, 0], dsb[ss], slsem[ss]).wait()

    def start_gather(ss, j, b):
        pltpu.async_copy(table.at[ssb[ss].at[j]], rows[b], gsem[b])

    def wait_gather(ss, j, b):
        pltpu.make_async_copy(table.at[ssb[ss].at[j]], rows[b], gsem[b]).wait()

    def drain_scatter(ss, j, b):
        pltpu.make_async_copy(rows[b], acc.at[dsb[ss].at[j]], ssem[b]).wait()
        if with_counts:
            pltpu.make_async_copy(onesv, cntacc.at[dsb[ss].at[j]],
                                  ssem[b]).wait()

    def chunk(k, ss, j):
        b = j % RING
        nb = (b + 1) % RING
        ss1, j1 = (ss, j + 1) if j + 1 < SLAB else (1 - ss, 0)
        ssp, jp = (ss, j - 1) if j > 0 else (1 - ss, SLAB - 1)
        wait_gather(ss, j, b)

        @pl.when(k >= 1)
        def _():
            drain_scatter(ssp, jp, nb)

        @pl.when(k + 1 < NCHUNK)
        def _():
            if j1 == 0:
                wait_slab(ss1)
            start_gather(ss1, j1, nb)

        pltpu.async_copy(rows[b], acc.at[dsb[ss].at[j]], ssem[b], add=True)
        if with_counts:
            pltpu.async_copy(onesv, cntacc.at[dsb[ss].at[j]], ssem[b],
                             add=True)

        if j == 1:
            @pl.when((k >= SLAB) & (k < (NSLAB - 1) * SLAB))
            def _():
                load_slab(k // SLAB + 1, 1 - ss)

    load_slab(0, 0)
    load_slab(1, 1)
    wait_slab(0)
    start_gather(0, 0, 0)

    @pl.loop(0, NSLAB // 2)
    def _step(i):
        for ss in range(2):
            for j in range(SLAB):
                chunk((2 * i + ss) * SLAB + j, ss, j)

    drain_scatter(1, SLAB - 1, (SLAB - 1) % RING)

    plsc.subcore_barrier()

    pltpu.sync_copy(acc.at[pl.ds(s * ROWS_PT, ROWS_PT)],
                    out.at[c, pl.ds(s * ROWS_PT, ROWS_PT)])
    if with_counts:
        pltpu.sync_copy(cntacc.at[pl.ds(s * ROWS_PT, ROWS_PT)],
                        cnt.at[pl.ds(c * NPAD + s * ROWS_PT, ROWS_PT)])


def _make_seg(with_counts):
    out_type = [jax.ShapeDtypeStruct((NC, NPAD, D), jnp.float32)]
    scratch = [
        pltpu.VMEM_SHARED((NPAD, D), jnp.float32),
    ]
    if with_counts:
        out_type.append(jax.ShapeDtypeStruct((NC * NPAD,), jnp.float32))
        scratch.append(pltpu.VMEM_SHARED((NPAD,), jnp.float32))
    scratch += [pltpu.VMEM((SLAB, CH), jnp.int32) for _ in range(4)]
    scratch += [pltpu.VMEM((CH, D), jnp.float32) for _ in range(RING)]
    if with_counts:
        scratch += [pltpu.VMEM((CH,), jnp.float32),
                    pltpu.VMEM((ROWS_PT,), jnp.float32)]
    scratch += [pltpu.SemaphoreType.DMA for _ in range(2 * RING + 2)]
    return pl.kernel(
        functools.partial(_seg_body, with_counts),
        out_type=out_type,
        mesh=plsc.VectorSubcoreMesh(core_axis_name="c", subcore_axis_name="s",
                                    num_cores=NC, num_subcores=NS),
        scratch_types=scratch,
    )


_make_seg = functools.cache(_make_seg)

BR = 2000
_HI = jax.lax.Precision.DEFAULT


def _dot_t(x, w):
    return jax.lax.dot_general(x, w, (((1,), (1,)), ((), ())), precision=_HI)


def _mm_body(x_ref, w_ref, b_ref, o_ref):
    o_ref[...] = _dot_t(x_ref[...], w_ref[...]) + b_ref[...]


def _mid_body(p_ref, cnt_ref, xr_ref, w_ref, b_ref, o_ref):
    sblk = p_ref[0] + p_ref[1]
    c2 = cnt_ref[...]
    inv = 1.0 / jnp.maximum(c2[:, 0:1] + c2[:, 1:2], 1.0)
    h = jnp.maximum(sblk * inv + xr_ref[...], 0.0)
    o_ref[...] = _dot_t(h, w_ref[...]) + b_ref[...]


def _fin_body(q_ref, cnt_ref, hr_ref, o_ref):
    sblk = q_ref[0] + q_ref[1]
    c2 = cnt_ref[...]
    inv = 1.0 / jnp.maximum(c2[:, 0:1] + c2[:, 1:2], 1.0)
    o_ref[...] = sblk * inv + hr_ref[...]


_row_spec = pl.BlockSpec((BR, D), lambda i: (i, 0))
_p_spec = pl.BlockSpec((NC, BR, D), lambda i: (0, i, 0))
_cnt_spec = pl.BlockSpec((BR, 2), lambda i: (i, 0))
_w_spec = pl.BlockSpec((D, D), lambda i: (0, 0))
_b_spec = pl.BlockSpec((1, D), lambda i: (0, 0))

_mm = pl.pallas_call(
    _mm_body,
    grid=(N // BR,),
    in_specs=[_row_spec, _w_spec, _b_spec],
    out_specs=_row_spec,
    out_shape=jax.ShapeDtypeStruct((N, D), jnp.float32),
)

_mid = pl.pallas_call(
    _mid_body,
    grid=(N // BR,),
    in_specs=[_p_spec, _cnt_spec, _row_spec, _w_spec, _b_spec],
    out_specs=_row_spec,
    out_shape=jax.ShapeDtypeStruct((N, D), jnp.float32),
)

_fin = pl.pallas_call(
    _fin_body,
    grid=(N // BR,),
    in_specs=[_p_spec, _cnt_spec, _row_spec],
    out_specs=_row_spec,
    out_shape=jax.ShapeDtypeStruct((N, D), jnp.float32),
)


def kernel(x, edge_index, W1_l, b1, W1_r, W2_l, b2, W2_r):
    ei = edge_index.astype(jnp.int32).reshape(2, NW, EPW)
    npe = EPWP - EPW
    ar = jnp.arange(NW * npe, dtype=jnp.int32)
    pad = jnp.stack([(ar % N).reshape(NW, npe),
                     (N + ar % (NPAD - N)).reshape(NW, npe)])
    edges = jnp.concatenate([ei, pad], axis=2).reshape(2, NW, NSLAB, SLAB, CH)
    z = jnp.zeros((1, D), jnp.float32)
    xl1 = _mm(x, W1_l, z)
    p, cnt = _make_seg(True)(xl1, edges)
    xr1 = _mm(x, W1_r, b1.reshape(1, D))
    cntT = cnt.reshape(NC, NPAD)[:, :N].T
    hl2 = _mid(p, cntT, xr1, W2_l, z)
    q, = _make_seg(False)(hl2, edges)
    hr2 = _mid(p, cntT, xr1, W2_r, b2.reshape(1, D))
    return _fin(q, cntT, hr2)
